# baseline (speedup 1.0000x reference)
import functools

import jax
import jax.numpy as jnp
from jax import lax
from jax.experimental import pallas as pl
from jax.experimental.pallas import tpu as pltpu
from jax.experimental.pallas import tpu_sc as plsc

_VOCAB_ROWS = 262144
_D = 1024
_B = 4 * 8192
_NC = 2
_NS = 16
_NW = _NC * _NS
_BPW = _B // _NW
_C = 32
_NCHUNK = _BPW // _C
_SCALE = 32.0
_LANES = 16


def _emb_body(tokens_hbm, table_hbm, out_hbm, idx_v, buf, sem):
    wid = lax.axis_index("s") * _NC + lax.axis_index("c")
    base = wid * _BPW
    pltpu.sync_copy(tokens_hbm.at[pl.ds(base, _BPW)], idx_v)

    def chunk(i, carry):
        cb = i * _C
        pltpu.async_copy(
            table_hbm.at[idx_v.at[pl.ds(cb, _C)]], buf, sem
        ).wait()

        def row(r, c2):
            for j in range(_D // _LANES):
                s = pl.ds(j * _LANES, _LANES)
                buf[r, s] = buf[r, s] * _SCALE
            return c2

        lax.fori_loop(0, _C, row, 0)
        pltpu.sync_copy(buf, out_hbm.at[pl.ds(base + cb, _C)])
        return carry

    lax.fori_loop(0, _NCHUNK, chunk, 0)


@jax.jit
def _emb_call(tokens_flat, table):
    mesh = plsc.VectorSubcoreMesh(core_axis_name="c", subcore_axis_name="s")
    k = functools.partial(
        pl.kernel,
        mesh=mesh,
        out_type=jax.ShapeDtypeStruct((_B, _D), jnp.float32),
        scratch_types=[
            pltpu.VMEM((_BPW,), jnp.int32),
            pltpu.VMEM((_C, _D), jnp.float32),
            pltpu.SemaphoreType.DMA,
        ],
    )(_emb_body)
    return k(tokens_flat, table)


def kernel(tokens, input_embedding):
    tok_flat = tokens.reshape(-1)
    out = _emb_call(tok_flat, input_embedding)
    return out.reshape(tokens.shape[0], tokens.shape[1], _D)


# speedup vs baseline: 1.7533x; 1.7533x over previous
import functools

import jax
import jax.numpy as jnp
from jax import lax
from jax.experimental import pallas as pl
from jax.experimental.pallas import tpu as pltpu
from jax.experimental.pallas import tpu_sc as plsc

_D = 1024
_B = 4 * 8192
_NC = 2
_NS = 16
_NW = _NC * _NS
_BPW = _B // _NW
_C = 32
_NCHUNK = _BPW // _C
_NBUF = 3
_SCALE = 32.0
_LANES = 16


def _mul_chunk(buf):
    def row(r, c2):
        for j in range(_D // _LANES):
            s = pl.ds(j * _LANES, _LANES)
            buf[r, s] = buf[r, s] * _SCALE
        return c2

    lax.fori_loop(0, _C, row, 0, unroll=False)


def _emb_body(tokens_hbm, table_hbm, out_hbm, idx_v,
              b0, b1, b2, i0, i1, i2, o0, o1, o2):
    bufs = (b0, b1, b2)
    isems = (i0, i1, i2)
    osems = (o0, o1, o2)
    wid = lax.axis_index("s") * _NC + lax.axis_index("c")
    base = wid * _BPW
    pltpu.sync_copy(tokens_hbm.at[pl.ds(base, _BPW)], idx_v)

    def gather_start(c, b):
        pltpu.async_copy(
            table_hbm.at[idx_v.at[pl.ds(c * _C, _C)]], bufs[b], isems[b]
        )

    def gather_wait(c, b):
        pltpu.make_async_copy(
            table_hbm.at[idx_v.at[pl.ds(c * _C, _C)]], bufs[b], isems[b]
        ).wait()

    def scatter_start(c, b):
        pltpu.async_copy(
            bufs[b], out_hbm.at[pl.ds(base + c * _C, _C)], osems[b]
        )

    def scatter_wait(c, b):
        pltpu.make_async_copy(
            bufs[b], out_hbm.at[pl.ds(base + c * _C, _C)], osems[b]
        ).wait()

    gather_start(0, 0)
    gather_start(1, 1)

    gather_wait(0, 0)
    _mul_chunk(b0)
    scatter_start(0, 0)
    gather_start(2, 2)
    gather_wait(1, 1)
    _mul_chunk(b1)
    scatter_start(1, 1)
    scatter_wait(0, 0)
    gather_start(3, 0)
    gather_wait(2, 2)
    _mul_chunk(b2)
    scatter_start(2, 2)
    scatter_wait(1, 1)
    gather_start(4, 1)

    def step(g, carry):
        i = 3 + g * _NBUF
        for bi in range(_NBUF):
            c = i + bi
            b = bi
            b2n = (bi + 2) % _NBUF
            gather_wait(c, b)
            _mul_chunk(bufs[b])
            scatter_start(c, b)
            scatter_wait(c - 1, b2n)
            gather_start(c + 2, b2n)
        return carry

    lax.fori_loop(0, (_NCHUNK - 5) // _NBUF, step, 0, unroll=False)

    gather_wait(_NCHUNK - 2, 0)
    _mul_chunk(b0)
    scatter_start(_NCHUNK - 2, 0)
    gather_wait(_NCHUNK - 1, 1)
    _mul_chunk(b1)
    scatter_start(_NCHUNK - 1, 1)
    scatter_wait(_NCHUNK - 3, 2)
    scatter_wait(_NCHUNK - 2, 0)
    scatter_wait(_NCHUNK - 1, 1)


@jax.jit
def _emb_call(tokens_flat, table):
    mesh = plsc.VectorSubcoreMesh(core_axis_name="c", subcore_axis_name="s")
    k = functools.partial(
        pl.kernel,
        mesh=mesh,
        out_type=jax.ShapeDtypeStruct((_B, _D), jnp.float32),
        scratch_types=[
            pltpu.VMEM((_BPW,), jnp.int32),
            pltpu.VMEM((_C, _D), jnp.float32),
            pltpu.VMEM((_C, _D), jnp.float32),
            pltpu.VMEM((_C, _D), jnp.float32),
            pltpu.SemaphoreType.DMA,
            pltpu.SemaphoreType.DMA,
            pltpu.SemaphoreType.DMA,
            pltpu.SemaphoreType.DMA,
            pltpu.SemaphoreType.DMA,
            pltpu.SemaphoreType.DMA,
        ],
    )(_emb_body)
    return k(tokens_flat, table)


def kernel(tokens, input_embedding):
    tok_flat = tokens.reshape(-1)
    out = _emb_call(tok_flat, input_embedding)
    return out.reshape(tokens.shape[0], tokens.shape[1], _D)


# speedup vs baseline: 1.7699x; 1.0095x over previous
import functools

import jax
import jax.numpy as jnp
from jax import lax
from jax.experimental import pallas as pl
from jax.experimental.pallas import tpu as pltpu
from jax.experimental.pallas import tpu_sc as plsc

_D = 1024
_B = 4 * 8192
_NC = 2
_NS = 16
_NW = _NC * _NS
_BPW = _B // _NW
_C = 16
_NCHUNK = _BPW // _C
_NBUF = 4
_SCALE = 32.0
_LANES = 16


def _mul_chunk(buf):
    def row(r, c2):
        for j in range(_D // _LANES):
            s = pl.ds(j * _LANES, _LANES)
            buf[r, s] = buf[r, s] * _SCALE
        return c2

    lax.fori_loop(0, _C, row, 0, unroll=False)


def _emb_body(tokens_hbm, table_hbm, out_hbm, idx_v,
              b0, b1, b2, b3, i0, i1, i2, i3, o0, o1, o2, o3):
    bufs = (b0, b1, b2, b3)
    isems = (i0, i1, i2, i3)
    osems = (o0, o1, o2, o3)
    wid = lax.axis_index("s") * _NC + lax.axis_index("c")
    base = wid * _BPW
    pltpu.sync_copy(tokens_hbm.at[pl.ds(base, _BPW)], idx_v)

    def gather_start(c, b):
        pltpu.async_copy(
            table_hbm.at[idx_v.at[pl.ds(c * _C, _C)]], bufs[b], isems[b]
        )

    def gather_wait(c, b):
        pltpu.make_async_copy(
            table_hbm.at[idx_v.at[pl.ds(c * _C, _C)]], bufs[b], isems[b]
        ).wait()

    def scatter_start(c, b):
        pltpu.async_copy(
            bufs[b], out_hbm.at[pl.ds(base + c * _C, _C)], osems[b]
        )

    def scatter_wait(c, b):
        pltpu.make_async_copy(
            bufs[b], out_hbm.at[pl.ds(base + c * _C, _C)], osems[b]
        ).wait()

    def steady_visit(c, bi):
        b2n = (bi + 2) % _NBUF
        gather_wait(c, bi)
        scatter_wait(c - 2, b2n)
        gather_start(c + 2, b2n)
        _mul_chunk(bufs[bi])
        scatter_start(c, bi)

    gather_start(0, 0)
    gather_start(1, 1)

    gather_wait(0, 0)
    gather_start(2, 2)
    _mul_chunk(b0)
    scatter_start(0, 0)
    gather_wait(1, 1)
    gather_start(3, 3)
    _mul_chunk(b1)
    scatter_start(1, 1)
    steady_visit(2, 2)
    steady_visit(3, 3)

    def step(g, carry):
        i = 4 + g * _NBUF
        for bi in range(_NBUF):
            steady_visit(i + bi, bi)
        return carry

    lax.fori_loop(0, (_NCHUNK - 8) // _NBUF, step, 0, unroll=False)

    steady_visit(_NCHUNK - 4, 0)
    steady_visit(_NCHUNK - 3, 1)

    gather_wait(_NCHUNK - 2, 2)
    _mul_chunk(b2)
    scatter_start(_NCHUNK - 2, 2)
    gather_wait(_NCHUNK - 1, 3)
    _mul_chunk(b3)
    scatter_start(_NCHUNK - 1, 3)
    scatter_wait(_NCHUNK - 4, 0)
    scatter_wait(_NCHUNK - 3, 1)
    scatter_wait(_NCHUNK - 2, 2)
    scatter_wait(_NCHUNK - 1, 3)


@jax.jit
def _emb_call(tokens_flat, table):
    mesh = plsc.VectorSubcoreMesh(core_axis_name="c", subcore_axis_name="s")
    k = functools.partial(
        pl.kernel,
        mesh=mesh,
        out_type=jax.ShapeDtypeStruct((_B, _D), jnp.float32),
        scratch_types=[
            pltpu.VMEM((_BPW,), jnp.int32),
            pltpu.VMEM((_C, _D), jnp.float32),
            pltpu.VMEM((_C, _D), jnp.float32),
            pltpu.VMEM((_C, _D), jnp.float32),
            pltpu.VMEM((_C, _D), jnp.float32),
            pltpu.SemaphoreType.DMA,
            pltpu.SemaphoreType.DMA,
            pltpu.SemaphoreType.DMA,
            pltpu.SemaphoreType.DMA,
            pltpu.SemaphoreType.DMA,
            pltpu.SemaphoreType.DMA,
            pltpu.SemaphoreType.DMA,
            pltpu.SemaphoreType.DMA,
        ],
    )(_emb_body)
    return k(tokens_flat, table)


def kernel(tokens, input_embedding):
    tok_flat = tokens.reshape(-1)
    out = _emb_call(tok_flat, input_embedding)
    return out.reshape(tokens.shape[0], tokens.shape[1], _D)
